# baseline (speedup 1.0000x reference)
import functools

import jax
import jax.numpy as jnp
from jax import lax
from jax.experimental import pallas as pl
from jax.experimental.pallas import tpu as pltpu
from jax.experimental.pallas import tpu_sc as plsc

L = 16
NC = 2
NS = 16
NW = NC * NS


def _build(rows, feats):
    rpw = rows // NW
    rb = 8
    nb = rpw // rb
    nch = feats // L

    mesh = plsc.VectorSubcoreMesh(core_axis_name="c", subcore_axis_name="s")

    @functools.partial(
        pl.kernel,
        out_type=(
            jax.ShapeDtypeStruct((rows, feats), jnp.float32),
            jax.ShapeDtypeStruct((rows,), jnp.float32),
        ),
        mesh=mesh,
        scratch_types=[
            pltpu.VMEM((3, rb, feats), jnp.float32),
            pltpu.VMEM((rpw,), jnp.float32),
            pltpu.SemaphoreType.DMA,
            pltpu.SemaphoreType.DMA,
            pltpu.SemaphoreType.DMA,
            pltpu.SemaphoreType.DMA,
            pltpu.SemaphoreType.DMA,
            pltpu.SemaphoreType.DMA,
        ],
    )
    def rev_kernel(x_hbm, y_hbm, ld_hbm, buf, zeros_v,
                   sin0, sin1, sin2, sout0, sout1, sout2):
        wid = lax.axis_index("s") * NC + lax.axis_index("c")
        base = wid * rpw
        sins = (sin0, sin1, sin2)
        souts = (sout0, sout1, sout2)

        zv = jnp.zeros((L,), jnp.float32)

        @plsc.parallel_loop(0, rpw // L)
        def _zfill(i):
            zeros_v[pl.ds(i * L, L)] = zv

        pltpu.sync_copy(zeros_v, ld_hbm.at[pl.ds(base, rpw)])

        def in_copy(g, b):
            return pltpu.async_copy(
                x_hbm.at[pl.ds(base + g * rb, rb)], buf.at[b], sins[b])

        def wait_in(g, b):
            pltpu.make_async_copy(
                x_hbm.at[pl.ds(base + g * rb, rb)], buf.at[b], sins[b]).wait()

        def out_copy(g, b):
            return pltpu.async_copy(
                buf.at[b], y_hbm.at[pl.ds(base + g * rb, rb)], souts[b])

        def wait_out(g, b):
            pltpu.make_async_copy(
                buf.at[b], y_hbm.at[pl.ds(base + g * rb, rb)],
                souts[b]).wait()

        def compute(b):
            for r in range(rb):
                @plsc.parallel_loop(0, nch // 2, unroll=8)
                def _pair(j):
                    lo = buf[b, r, pl.ds(j * L, L)]
                    hi = buf[b, r, pl.ds((nch - 1 - j) * L, L)]
                    buf[b, r, pl.ds(j * L, L)] = lax.rev(hi, (0,))
                    buf[b, r, pl.ds((nch - 1 - j) * L, L)] = lax.rev(lo, (0,))

        in_copy(0, 0)
        for g in range(2):
            wait_in(g, g)
            in_copy(g + 1, g + 1)
            compute(g)
            out_copy(g, g)

        @pl.loop(2, nb, step=3)
        def _blocks(g0):
            for k in range(3):
                b = (2 + k) % 3
                g = g0 + k
                wait_in(g, b)
                wait_out(g - 2, (b + 1) % 3)

                @pl.when(g + 1 < nb)
                def _prefetch():
                    in_copy(g + 1, (b + 1) % 3)

                compute(b)
                out_copy(g, b)

        for g in range(nb - 2, nb):
            wait_out(g, g % 3)

    return rev_kernel


def kernel(x, perm):
    rows, feats = x.shape
    y, logdet = _build(rows, feats)(x)
    return (y, logdet)


# speedup vs baseline: 1.0361x; 1.0361x over previous
import functools

import jax
import jax.numpy as jnp
from jax import lax
from jax.experimental import pallas as pl
from jax.experimental.pallas import tpu as pltpu
from jax.experimental.pallas import tpu_sc as plsc

L = 16
NC = 2
NS = 16
NW = NC * NS


def _build(rows, feats):
    rpw = rows // NW
    rbi = 8
    rbo = 4
    nb = rpw // rbi
    nch = feats // L

    mesh = plsc.VectorSubcoreMesh(core_axis_name="c", subcore_axis_name="s")

    @functools.partial(
        pl.kernel,
        out_type=(
            jax.ShapeDtypeStruct((rows, feats), jnp.float32),
            jax.ShapeDtypeStruct((rows,), jnp.float32),
        ),
        mesh=mesh,
        scratch_types=[
            pltpu.VMEM((2, rbi, feats), jnp.float32),
            pltpu.VMEM((2, rbo, feats), jnp.float32),
            pltpu.VMEM((rpw,), jnp.float32),
            pltpu.SemaphoreType.DMA,
            pltpu.SemaphoreType.DMA,
            pltpu.SemaphoreType.DMA,
            pltpu.SemaphoreType.DMA,
        ],
    )
    def rev_kernel(x_hbm, y_hbm, ld_hbm, in_v, out_v, zeros_v,
                   sin0, sin1, sout0, sout1):
        wid = lax.axis_index("s") * NC + lax.axis_index("c")
        base = wid * rpw
        sins = (sin0, sin1)
        souts = (sout0, sout1)

        zv = jnp.zeros((L,), jnp.float32)

        @plsc.parallel_loop(0, rpw // L)
        def _zfill(i):
            zeros_v[pl.ds(i * L, L)] = zv

        pltpu.sync_copy(zeros_v, ld_hbm.at[pl.ds(base, rpw)])

        def in_copy(g, b):
            return pltpu.async_copy(
                x_hbm.at[pl.ds(base + g * rbi, rbi)], in_v.at[b], sins[b])

        def out_copy(g, h):
            return pltpu.async_copy(
                out_v.at[h],
                y_hbm.at[pl.ds(base + g * rbi + h * rbo, rbo)], souts[h])

        in_copy(0, 0)

        @pl.loop(0, nb, step=2)
        def _blocks(g0):
            for b in range(2):
                g = g0 + b
                bn = (b + 1) % 2

                @pl.when(g + 1 < nb)
                def _prefetch():
                    in_copy(g + 1, bn)

                pltpu.make_async_copy(
                    x_hbm.at[pl.ds(base + g * rbi, rbi)],
                    in_v.at[b], sins[b]).wait()

                for h in range(2):
                    @pl.when(g >= 1)
                    def _drain():
                        pltpu.make_async_copy(
                            out_v.at[h],
                            y_hbm.at[pl.ds(base + g * rbi + h * rbo, rbo)],
                            souts[h]).wait()

                    for r in range(rbo):
                        @plsc.parallel_loop(0, nch, unroll=8)
                        def _chunk(j):
                            v = in_v[b, h * rbo + r,
                                     pl.ds((nch - 1 - j) * L, L)]
                            out_v[h, r, pl.ds(j * L, L)] = lax.rev(v, (0,))

                    out_copy(g, h)

        for h in range(2):
            pltpu.make_async_copy(
                out_v.at[h],
                y_hbm.at[pl.ds(base + (nb - 1) * rbi + h * rbo, rbo)],
                souts[h]).wait()

    return rev_kernel


def kernel(x, perm):
    rows, feats = x.shape
    y, logdet = _build(rows, feats)(x)
    return (y, logdet)
